# baseline (speedup 1.0000x reference)
import functools

import jax
import jax.numpy as jnp
from jax import lax
from jax.experimental import pallas as pl
from jax.experimental.pallas import tpu as pltpu
from jax.experimental.pallas import tpu_sc as plsc

_ROWS, _DIM = 8192, 1024
_NC, _NS = 2, 16
_NW = _NC * _NS
_RPW = _ROWS // _NW


def _copy_body(table, out, sem):
    wid = lax.axis_index("s") * _NC + lax.axis_index("c")
    base = wid * _RPW
    copy = pltpu.make_async_copy(
        table.at[pl.ds(base, _RPW)], out.at[pl.ds(base, _RPW)], sem
    )
    copy.start()
    copy.wait()


@functools.partial(jax.jit, static_argnums=())
def _sc_copy(emb_weight):
    mesh = plsc.VectorSubcoreMesh(core_axis_name="c", subcore_axis_name="s")
    return pl.kernel(
        _copy_body,
        out_type=jax.ShapeDtypeStruct((_ROWS, _DIM), jnp.float32),
        mesh=mesh,
        scratch_types=[pltpu.SemaphoreType.DMA],
    )(emb_weight)


def kernel(x, emb_weight):
    del x
    return _sc_copy(emb_weight)


# speedup vs baseline: 1.0160x; 1.0160x over previous
import jax
import jax.numpy as jnp
from jax.experimental import pallas as pl
from jax.experimental.pallas import tpu as pltpu

_ROWS, _DIM = 8192, 1024
_NCHUNKS = 16
_CROWS = _ROWS // _NCHUNKS


def _copy_body(w_ref, o_ref, sems):
    for i in range(_NCHUNKS):
        pltpu.make_async_copy(
            w_ref.at[pl.ds(i * _CROWS, _CROWS)],
            o_ref.at[pl.ds(i * _CROWS, _CROWS)],
            sems.at[i],
        ).start()
    for i in range(_NCHUNKS):
        pltpu.make_async_copy(
            w_ref.at[pl.ds(i * _CROWS, _CROWS)],
            o_ref.at[pl.ds(i * _CROWS, _CROWS)],
            sems.at[i],
        ).wait()


def kernel(x, emb_weight):
    del x
    return pl.pallas_call(
        _copy_body,
        out_shape=jax.ShapeDtypeStruct(emb_weight.shape, emb_weight.dtype),
        in_specs=[pl.BlockSpec(memory_space=pltpu.MemorySpace.HBM)],
        out_specs=pl.BlockSpec(memory_space=pltpu.MemorySpace.HBM),
        scratch_shapes=[pltpu.SemaphoreType.DMA((_NCHUNKS,))],
    )(emb_weight)


# speedup vs baseline: 24.2906x; 23.9073x over previous
import functools

import jax
import jax.numpy as jnp
from jax import lax
from jax.experimental import pallas as pl
from jax.experimental.pallas import tpu as pltpu
from jax.experimental.pallas import tpu_sc as plsc

_ROWS, _DIM = 8192, 1024
_NC, _NS = 2, 16
_NW = _NC * _NS
_RPW = _ROWS // _NW
_NBUF = 4
_CHUNK = 16
_NCHUNK = _RPW // _CHUNK


def _copy_body(table, out, *refs):
    bufs = refs[:_NBUF]
    sin = refs[_NBUF : 2 * _NBUF]
    sout = refs[2 * _NBUF : 3 * _NBUF]
    wid = lax.axis_index("s") * _NC + lax.axis_index("c")
    base = wid * _RPW

    def load(c):
        b = c % _NBUF
        return pltpu.make_async_copy(
            table.at[pl.ds(base + c * _CHUNK, _CHUNK)], bufs[b], sin[b]
        )

    def store(c):
        b = c % _NBUF
        return pltpu.make_async_copy(
            bufs[b], out.at[pl.ds(base + c * _CHUNK, _CHUNK)], sout[b]
        )

    for c in range(_NBUF):
        load(c).start()
    for c in range(_NCHUNK):
        load(c).wait()
        store(c).start()
        nxt = c + _NBUF
        if nxt < _NCHUNK:
            store(c).wait()
            load(nxt).start()
    for c in range(_NCHUNK - _NBUF, _NCHUNK):
        store(c).wait()


@jax.jit
def _sc_copy(emb_weight):
    mesh = plsc.VectorSubcoreMesh(core_axis_name="c", subcore_axis_name="s")
    scratch = [pltpu.VMEM((_CHUNK, _DIM), jnp.float32) for _ in range(_NBUF)]
    scratch += [pltpu.SemaphoreType.DMA for _ in range(2 * _NBUF)]
    return pl.kernel(
        _copy_body,
        out_type=jax.ShapeDtypeStruct((_ROWS, _DIM), jnp.float32),
        mesh=mesh,
        scratch_types=scratch,
    )(emb_weight)


def kernel(x, emb_weight):
    del x
    return _sc_copy(emb_weight)


# speedup vs baseline: 46.0369x; 1.8953x over previous
import jax
import jax.numpy as jnp
from jax.experimental import pallas as pl
from jax.experimental.pallas import tpu as pltpu

_BLOCK_ROWS = 1024


def _copy_block(w_ref, o_ref):
    o_ref[...] = w_ref[...]


def kernel(x, emb_weight):
    del x
    rows, dim = emb_weight.shape
    grid = (rows // _BLOCK_ROWS,)
    return pl.pallas_call(
        _copy_block,
        grid=grid,
        in_specs=[pl.BlockSpec((_BLOCK_ROWS, dim), lambda i: (i, 0))],
        out_specs=pl.BlockSpec((_BLOCK_ROWS, dim), lambda i: (i, 0)),
        out_shape=jax.ShapeDtypeStruct(emb_weight.shape, emb_weight.dtype),
    )(emb_weight)


# speedup vs baseline: 49.7893x; 1.0815x over previous
import jax
import jax.numpy as jnp
from jax.experimental import pallas as pl
from jax.experimental.pallas import tpu as pltpu

_BLOCK_ROWS = 2048


def _copy_block(w_ref, o_ref):
    o_ref[...] = w_ref[...]


def kernel(x, emb_weight):
    del x
    rows, dim = emb_weight.shape
    grid = (rows // _BLOCK_ROWS,)
    return pl.pallas_call(
        _copy_block,
        grid=grid,
        in_specs=[pl.BlockSpec((_BLOCK_ROWS, dim), lambda i: (i, 0))],
        out_specs=pl.BlockSpec((_BLOCK_ROWS, dim), lambda i: (i, 0)),
        out_shape=jax.ShapeDtypeStruct(emb_weight.shape, emb_weight.dtype),
    )(emb_weight)


# speedup vs baseline: 50.8243x; 1.0208x over previous
import jax
import jax.numpy as jnp
from jax.experimental import pallas as pl
from jax.experimental.pallas import tpu as pltpu

_ROWS, _DIM = 8192, 1024
_CHUNK = 1024
_NBUF = 6
_NCHUNK = _ROWS // _CHUNK


def _copy_body(w_ref, o_ref, *refs):
    bufs = refs[:_NBUF]
    sin = refs[_NBUF]
    sout = refs[_NBUF + 1]

    def load(c):
        b = c % _NBUF
        return pltpu.make_async_copy(
            w_ref.at[pl.ds(c * _CHUNK, _CHUNK)], bufs[b], sin.at[b]
        )

    def store(c):
        b = c % _NBUF
        return pltpu.make_async_copy(
            bufs[b], o_ref.at[pl.ds(c * _CHUNK, _CHUNK)], sout.at[b]
        )

    for c in range(min(_NBUF, _NCHUNK)):
        load(c).start()
    for c in range(_NCHUNK):
        load(c).wait()
        store(c).start()
        nxt = c + _NBUF
        if nxt < _NCHUNK:
            store(c).wait()
            load(nxt).start()
    for c in range(max(_NCHUNK - _NBUF, 0), _NCHUNK):
        store(c).wait()


def kernel(x, emb_weight):
    del x
    return pl.pallas_call(
        _copy_body,
        out_shape=jax.ShapeDtypeStruct(emb_weight.shape, emb_weight.dtype),
        in_specs=[pl.BlockSpec(memory_space=pltpu.MemorySpace.HBM)],
        out_specs=pl.BlockSpec(memory_space=pltpu.MemorySpace.HBM),
        scratch_shapes=[pltpu.VMEM((_CHUNK, _DIM), jnp.float32) for _ in range(_NBUF)]
        + [pltpu.SemaphoreType.DMA((_NBUF,)), pltpu.SemaphoreType.DMA((_NBUF,))],
    )(emb_weight)
